# Initial kernel scaffold; baseline (speedup 1.0000x reference)
import functools

import jax
import jax.numpy as jnp
from jax import lax
from jax.experimental import pallas as pl
from jax.experimental.pallas import tpu as pltpu
from jax.experimental.pallas import tpu_sc as plsc

NC = 2
NS = 16
NW = NC * NS
L = 16

CH = 4


def _sigmoid(x):
    return 1.0 / (1.0 + jnp.exp(-x))


@functools.partial(jax.jit, static_argnums=(4, 5, 6))
def _run(ids2, table, w, b, batch, hist, half):
    bags_per_w = batch // NW
    n_chunks = bags_per_w // CH
    rows_per_chunk = CH * hist
    d = table.shape[1]
    nk = d // L

    dstidx = jnp.repeat(jnp.arange(CH, dtype=jnp.int32), 2)[:, None] * jnp.ones(
        (1, half), dtype=jnp.int32)

    mesh = plsc.VectorSubcoreMesh(core_axis_name="c", subcore_axis_name="s")

    @functools.partial(
        pl.kernel,
        out_type=jax.ShapeDtypeStruct((batch,), jnp.float32),
        mesh=mesh,
        scratch_types=[
            pltpu.VMEM((2 * CH, half), jnp.int32),
            pltpu.VMEM((rows_per_chunk, d), jnp.float32),
            pltpu.VMEM((CH, d), jnp.float32),
            pltpu.VMEM((2 * CH, half), jnp.int32),
            pltpu.VMEM((d,), jnp.float32),
            pltpu.VMEM((1,), jnp.float32),
            pltpu.VMEM((bags_per_w,), jnp.float32),
            pltpu.SemaphoreType.DMA,
        ],
    )
    def k(ids2_hbm, table_hbm, w_hbm, b_hbm, dstidx_hbm, out_hbm,
          idx_v, stage_v, acc_v, dstidx_v, w_v, b_v, logit_v, gsem):
        wid = lax.axis_index("s") * NC + lax.axis_index("c")
        base_bag = wid * bags_per_w

        pltpu.sync_copy(w_hbm.at[0], w_v)
        pltpu.sync_copy(b_hbm, b_v)
        pltpu.sync_copy(dstidx_hbm, dstidx_v)

        zero = jnp.zeros((L,), jnp.float32)
        inv_hist = jnp.float32(1.0 / hist)

        def chunk_body(m, carry):
            g0 = m * CH
            pltpu.sync_copy(
                ids2_hbm.at[pl.ds((base_bag + g0) * 2, 2 * CH)], idx_v)
            for s in range(CH):
                for kk in range(nk):
                    acc_v[s, pl.ds(kk * L, L)] = zero
            cps = [
                pltpu.async_copy(
                    table_hbm.at[idx_v.at[c]],
                    stage_v.at[pl.ds(c * half, half)], gsem)
                for c in range(2 * CH)
            ]
            for cp in cps:
                cp.wait()
            for c in range(2 * CH):
                pltpu.sync_copy(
                    stage_v.at[pl.ds(c * half, half)],
                    acc_v.at[dstidx_v.at[c]], add=True)
            b_s = b_v[0]
            for s in range(CH):
                p = zero
                for kk in range(nk):
                    e = acc_v[s, pl.ds(kk * L, L)] * inv_hist
                    p = p + _sigmoid(e) * w_v[pl.ds(kk * L, L)]
                logit_v[g0 + s] = jnp.sum(p) + b_s
            return carry

        lax.fori_loop(0, n_chunks, chunk_body, 0)

        for i in range(bags_per_w // L):
            logit_v[pl.ds(i * L, L)] = _sigmoid(logit_v[pl.ds(i * L, L)])
        pltpu.sync_copy(logit_v, out_hbm.at[pl.ds(base_bag, bags_per_w)])

    return k(ids2, table, w, b, dstidx)


def kernel(input_ids, emb_table, W, b):
    batch, hist = input_ids.shape
    assert hist % 2 == 0 and hist // 2 <= 128
    half = hist // 2
    ids2 = input_ids.reshape(batch * 2, half)
    out = _run(ids2, emb_table, W, b, batch, hist, half)
    return out.reshape(batch, 1)


# speedup vs baseline: 17.6558x; 17.6558x over previous
import functools

import jax
import jax.numpy as jnp
from jax import lax
from jax.experimental import pallas as pl
from jax.experimental.pallas import tpu as pltpu
from jax.experimental.pallas import tpu_sc as plsc

NC = 2
NS = 16
NW = NC * NS
L = 16

CH = 4


def _sigmoid(x):
    return 1.0 / (1.0 + jnp.exp(-x))


@functools.partial(jax.jit, static_argnums=(4, 5, 6))
def _run(ids2, table, w, b, batch, hist, half):
    bags_per_w = batch // NW
    n_chunks = bags_per_w // CH
    rows_per_chunk = CH * hist
    d = table.shape[1]
    nk = d // L

    mesh = plsc.VectorSubcoreMesh(core_axis_name="c", subcore_axis_name="s")
    group = L // CH

    @functools.partial(
        pl.kernel,
        out_type=jax.ShapeDtypeStruct((batch,), jnp.float32),
        mesh=mesh,
        compiler_params=pltpu.CompilerParams(
            needs_layout_passes=False, use_tc_tiling_on_sc=False),
        scratch_types=[
            pltpu.VMEM((2 * CH, half), jnp.int32),
            pltpu.VMEM((rows_per_chunk, d), jnp.float32),
            pltpu.VMEM((d,), jnp.float32),
            pltpu.VMEM((L,), jnp.float32),
            pltpu.VMEM((bags_per_w,), jnp.float32),
            pltpu.SemaphoreType.DMA,
        ],
    )
    def k(ids2_hbm, table_hbm, w_hbm, b_hbm, out_hbm,
          idx_v, stage_v, w_v, b_v, logit_v, gsem):
        wid = lax.axis_index("s") * NC + lax.axis_index("c")
        base_bag = wid * bags_per_w

        pltpu.sync_copy(w_hbm.at[0], w_v)
        pltpu.sync_copy(b_hbm, b_v)

        zero = jnp.zeros((L,), jnp.float32)
        inv_hist = jnp.float32(1.0 / hist)
        lanes = lax.iota(jnp.int32, L)
        wregs = tuple(w_v[pl.ds(kk * L, L)] for kk in range(nk))
        bvec = b_v[pl.ds(0, L)]

        def chunk_body(m, lvec):
            g0 = m * CH
            pltpu.sync_copy(
                ids2_hbm.at[pl.ds((base_bag + g0) * 2, 2 * CH)], idx_v)
            cps = [
                pltpu.async_copy(
                    table_hbm.at[idx_v.at[c]],
                    stage_v.at[pl.ds(c * half, half)], gsem)
                for c in range(2 * CH)
            ]
            for cp in cps:
                cp.wait()
            for s in range(CH):
                def row_body(j, accs):
                    r = s * hist + j
                    return tuple(
                        accs[kk] + stage_v[r, pl.ds(kk * L, L)]
                        for kk in range(nk))
                accs = lax.fori_loop(
                    0, hist, row_body, (zero,) * nk, unroll=4)
                p = zero
                for kk in range(nk):
                    e = accs[kk] * inv_hist
                    p = p + _sigmoid(e) * wregs[kk]
                lane = (m % group) * CH + s
                lvec = jnp.where(lanes == lane, jnp.sum(p), lvec)

            @pl.when(m % group == group - 1)
            def _():
                logit_v[pl.ds((m // group) * L, L)] = lvec

            return lvec

        lax.fori_loop(0, n_chunks, chunk_body, zero)

        for i in range(bags_per_w // L):
            logit_v[pl.ds(i * L, L)] = _sigmoid(
                logit_v[pl.ds(i * L, L)] + bvec)
        pltpu.sync_copy(logit_v, out_hbm.at[pl.ds(base_bag, bags_per_w)])

    return k(ids2, table, w, jnp.broadcast_to(b, (L,)))


def kernel(input_ids, emb_table, W, b):
    batch, hist = input_ids.shape
    assert hist % 2 == 0 and hist // 2 <= 128
    half = hist // 2
    ids2 = input_ids.reshape(batch * 2, half)
    out = _run(ids2, emb_table, W, b, batch, hist, half)
    return out.reshape(batch, 1)


# speedup vs baseline: 25.2426x; 1.4297x over previous
import functools

import jax
import jax.numpy as jnp
from jax import lax
from jax.experimental import pallas as pl
from jax.experimental.pallas import tpu as pltpu
from jax.experimental.pallas import tpu_sc as plsc

NC = 2
NS = 16
NW = NC * NS
L = 16

CH = 4


def _sigmoid(x):
    return 1.0 / (1.0 + jnp.exp(-x))


@functools.partial(jax.jit, static_argnums=(4, 5, 6))
def _run(ids2, table, w, b, batch, hist, half):
    bags_per_w = batch // NW
    n_chunks = bags_per_w // CH
    rows_per_chunk = CH * hist
    d = table.shape[1]
    nk = d // L

    mesh = plsc.VectorSubcoreMesh(core_axis_name="c", subcore_axis_name="s")
    group = L // CH

    @functools.partial(
        pl.kernel,
        out_type=jax.ShapeDtypeStruct((batch,), jnp.float32),
        mesh=mesh,
        compiler_params=pltpu.CompilerParams(
            needs_layout_passes=False, use_tc_tiling_on_sc=False),
        scratch_types=[
            pltpu.VMEM((2, 2 * CH, half), jnp.int32),
            pltpu.VMEM((2, rows_per_chunk, d), jnp.float32),
            pltpu.VMEM((d,), jnp.float32),
            pltpu.VMEM((L,), jnp.float32),
            pltpu.VMEM((bags_per_w,), jnp.float32),
            pltpu.SemaphoreType.DMA,
        ],
    )
    def k(ids2_hbm, table_hbm, w_hbm, b_hbm, out_hbm,
          idx_v, stage_v, w_v, b_v, logit_v, gsem):
        wid = lax.axis_index("s") * NC + lax.axis_index("c")
        base_bag = wid * bags_per_w

        pltpu.sync_copy(w_hbm.at[0], w_v)
        pltpu.sync_copy(b_hbm, b_v)

        zero = jnp.zeros((L,), jnp.float32)
        inv_hist = jnp.float32(1.0 / hist)
        lanes = lax.iota(jnp.int32, L)
        wregs = tuple(w_v[pl.ds(kk * L, L)] for kk in range(nk))
        bvec = b_v[pl.ds(0, L)]

        def fetch(m, buf):
            pltpu.sync_copy(
                ids2_hbm.at[pl.ds((base_bag + m * CH) * 2, 2 * CH)],
                idx_v.at[buf])
            for c in range(2 * CH):
                pltpu.async_copy(
                    table_hbm.at[idx_v.at[buf, c]],
                    stage_v.at[buf, pl.ds(c * half, half)], gsem)

        fetch(0, 0)

        def chunk_body(m, lvec):
            buf = lax.rem(m, 2)
            for c in range(2 * CH):
                pltpu.make_async_copy(
                    table_hbm.at[idx_v.at[buf, c]],
                    stage_v.at[buf, pl.ds(c * half, half)], gsem).wait()

            @pl.when(m + 1 < n_chunks)
            def _():
                fetch(m + 1, 1 - buf)

            for s in range(CH):
                def row_body(j, accs):
                    r = s * hist + j
                    return tuple(
                        accs[kk] + stage_v[buf, r, pl.ds(kk * L, L)]
                        for kk in range(nk))
                accs = lax.fori_loop(
                    0, hist, row_body, (zero,) * nk, unroll=4)
                p = zero
                for kk in range(nk):
                    e = accs[kk] * inv_hist
                    p = p + _sigmoid(e) * wregs[kk]
                lane = (m % group) * CH + s
                lvec = jnp.where(lanes == lane, jnp.sum(p), lvec)

            @pl.when(m % group == group - 1)
            def _():
                logit_v[pl.ds((m // group) * L, L)] = lvec

            return lvec

        lax.fori_loop(0, n_chunks, chunk_body, zero)

        for i in range(bags_per_w // L):
            logit_v[pl.ds(i * L, L)] = _sigmoid(
                logit_v[pl.ds(i * L, L)] + bvec)
        pltpu.sync_copy(logit_v, out_hbm.at[pl.ds(base_bag, bags_per_w)])

    return k(ids2, table, w, jnp.broadcast_to(b, (L,)))


def kernel(input_ids, emb_table, W, b):
    batch, hist = input_ids.shape
    assert hist % 2 == 0 and hist // 2 <= 128
    half = hist // 2
    ids2 = input_ids.reshape(batch * 2, half)
    out = _run(ids2, emb_table, W, b, batch, hist, half)
    return out.reshape(batch, 1)
